# baseline (speedup 1.0000x reference)
import jax
import jax.numpy as jnp
from jax import lax
from jax.experimental import pallas as pl
from jax.experimental.pallas import tpu as pltpu
from jax.experimental.pallas import tpu_sc as plsc

S = 2048
D = 768
E = 8
K = 2
H = 4 * D
A = S * K
BT = 256
NBP = A // BT + E - 1
PT = NBP * BT
NW = 32
CHW = A // NW
TKW = S // NW
CC = 256

_EPS = jax.random.normal(jax.random.key(42), (S, E), dtype=jnp.float32)
_EPS_T = _EPS.T


def _split64(v):
    hi = jnp.floor(v * (1.0 / 64.0))
    return hi, v - hi * 64.0


def _exact_dot_r(a, b):
    hi, lo = _split64(b)
    return (jnp.dot(a, hi, preferred_element_type=jnp.float32) * 64.0
            + jnp.dot(a, lo, preferred_element_type=jnp.float32))


def _router_kernel(zt_ref, epst_ref, z_ref, eps_ref,
                   pos_ref, gateb_ref, sched_ref):
    zt = zt_ref[...]
    noisyt = zt + epst_ref[...] * jax.nn.softplus(zt)
    idxe = lax.broadcasted_iota(jnp.int32, (E, S), 0)
    v0 = jnp.max(noisyt, axis=0, keepdims=True)
    i0 = jnp.min(jnp.where(noisyt == v0, idxe, E), axis=0, keepdims=True)
    m0 = idxe == i0
    masked = jnp.where(m0, -jnp.inf, noisyt)
    v1 = jnp.max(masked, axis=0, keepdims=True)
    i1 = jnp.min(jnp.where(masked == v1, idxe, E), axis=0, keepdims=True)
    m1 = idxe == i1

    oh0 = m0.astype(jnp.float32)
    oh1 = m1.astype(jnp.float32)
    counts_col = jnp.sum(oh0 + oh1, axis=1, keepdims=True)
    blocks_col = jnp.floor((counts_col + (BT - 1)) * (1.0 / BT))
    pad_col = blocks_col * BT
    tril = (lax.broadcasted_iota(jnp.int32, (E, E), 1)
            < lax.broadcasted_iota(jnp.int32, (E, E), 0)).astype(jnp.float32)
    offp_col = _exact_dot_r(tril, pad_col)

    up = (lax.broadcasted_iota(jnp.int32, (CC, CC), 0)
          < lax.broadcasted_iota(jnp.int32, (CC, CC), 1)).astype(jnp.float32)
    prefix = jnp.zeros((E, 1), jnp.float32)
    for slot, (oh, m) in enumerate(((oh0, m0), (oh1, m1))):
        for i in range(S // CC):
            blk = oh[:, i * CC:(i + 1) * CC]
            mblk = m[:, i * CC:(i + 1) * CC]
            rank = jnp.dot(blk, up, preferred_element_type=jnp.float32) + prefix
            dest = jnp.sum(jnp.where(mblk, rank + offp_col, 0.0),
                           axis=0, keepdims=True)
            pos_ref[slot:slot + 1, i * CC:(i + 1) * CC] = dest.astype(jnp.int32)
            prefix = prefix + jnp.sum(blk, axis=1, keepdims=True)

    z = z_ref[...]
    noisy = z + eps_ref[...] * jax.nn.softplus(z)
    idxe2 = lax.broadcasted_iota(jnp.int32, (S, E), 1)
    w0 = jnp.max(noisy, axis=1, keepdims=True)
    j0 = jnp.min(jnp.where(noisy == w0, idxe2, E), axis=1, keepdims=True)
    masked2 = jnp.where(idxe2 == j0, -jnp.inf, noisy)
    w1 = jnp.max(masked2, axis=1, keepdims=True)
    t = jnp.exp(w1 - w0)
    gateb_ref[0:S, :] = jnp.broadcast_to(1.0 / (1.0 + t), (S, 16))
    gateb_ref[S:2 * S, :] = jnp.broadcast_to(t / (1.0 + t), (S, 16))

    j1 = jnp.min(jnp.where(masked2 == w1, idxe2, E), axis=1, keepdims=True)
    counts_row = jnp.sum((idxe2 == j0).astype(jnp.float32)
                         + (idxe2 == j1).astype(jnp.float32),
                         axis=0, keepdims=True)
    blocks_row = jnp.floor((counts_row + (BT - 1)) * (1.0 / BT))
    triu_s = (lax.broadcasted_iota(jnp.int32, (E, E), 0)
              < lax.broadcasted_iota(jnp.int32, (E, E), 1)).astype(jnp.float32)
    bstart_row = jnp.dot(blocks_row, triu_s,
                         preferred_element_type=jnp.float32)
    total = jnp.sum(blocks_row, axis=1, keepdims=True)
    sched_ref[0:1, 0:E] = bstart_row.astype(jnp.int32)
    sched_ref[0:1, E:E + 1] = total.astype(jnp.int32)


def _gmm_kernel(sched_ref, xg_ref, w1_ref, b1_ref, w2_ref, b2_ref, out_ref):
    e = pl.program_id(0)
    lo = sched_ref[0, e]
    nb = sched_ref[0, e + 1] - lo

    def blk(j, carry):
        rb = (lo + j) * BT
        x = xg_ref[pl.ds(rb, BT), :]
        h = jnp.maximum(
            jnp.dot(x, w1_ref[0].astype(jnp.bfloat16),
                    preferred_element_type=jnp.float32)
            + b1_ref[0], 0.0).astype(jnp.bfloat16)
        o = jnp.dot(h, w2_ref[0].astype(jnp.bfloat16),
                    preferred_element_type=jnp.float32) + b2_ref[0]
        out_ref[pl.ds(rb, BT), :] = o.astype(jnp.bfloat16)
        return carry

    lax.fori_loop(0, nb, blk, 0)


def _dispatch_body(x_hbm, pos_hbm, xg_hbm, idx_v, xbuf, sem):
    c = lax.axis_index("c")
    sc = lax.axis_index("s")
    wid = sc * 2 + c
    tbase = (wid % 16) * CHW
    pltpu.sync_copy(x_hbm.at[pl.ds(tbase, CHW)], xbuf)
    pltpu.sync_copy(pos_hbm.at[pl.ds(wid * CHW, CHW)], idx_v)
    pltpu.async_copy(xbuf, xg_hbm.at[idx_v], sem).wait()


def _combine_body(y_hbm, pos_hbm, gateb_hbm, out_hbm,
                  i0_v, i1_v, g0_v, g1_v, buf0, buf1, sem):
    c = lax.axis_index("c")
    sc = lax.axis_index("s")
    wid = sc * 2 + c
    base = wid * TKW
    pltpu.sync_copy(pos_hbm.at[pl.ds(base, TKW)], i0_v)
    pltpu.sync_copy(pos_hbm.at[pl.ds(S + base, TKW)], i1_v)
    pltpu.sync_copy(gateb_hbm.at[pl.ds(base, TKW)], g0_v)
    pltpu.sync_copy(gateb_hbm.at[pl.ds(S + base, TKW)], g1_v)
    pltpu.async_copy(y_hbm.at[i0_v], buf0, sem).wait()
    pltpu.async_copy(y_hbm.at[i1_v], buf1, sem).wait()

    def row(r, carry):
        g0 = g0_v[r, pl.ds(0, 16)]
        g1 = g1_v[r, pl.ds(0, 16)]
        for j in range(D // 16):
            sl = pl.ds(j * 16, 16)
            buf0[r, sl] = buf0[r, sl] * g0 + buf1[r, sl] * g1
        return carry

    lax.fori_loop(0, TKW, row, 0)
    pltpu.sync_copy(buf0, out_hbm.at[pl.ds(base, TKW)])


def kernel(x, expert, W1, b1, W2, b2):
    flat_x = x.reshape(S, D)

    pos, gateb, sched = pl.pallas_call(
        _router_kernel,
        out_shape=[
            jax.ShapeDtypeStruct((K, S), jnp.int32),
            jax.ShapeDtypeStruct((K * S, 16), jnp.float32),
            jax.ShapeDtypeStruct((8, 128), jnp.int32),
        ],
    )(expert.T, _EPS_T, expert, _EPS)

    mesh = plsc.VectorSubcoreMesh(core_axis_name="c", subcore_axis_name="s")
    pos_flat = pos.reshape(A)

    xg = pl.kernel(
        _dispatch_body,
        out_type=jax.ShapeDtypeStruct((PT, D), jnp.float32),
        mesh=mesh,
        scratch_types=[
            pltpu.VMEM((CHW,), jnp.int32),
            pltpu.VMEM((CHW, D), jnp.float32),
            pltpu.SemaphoreType.DMA,
        ],
    )(flat_x, pos_flat)

    y = pl.pallas_call(
        _gmm_kernel,
        grid_spec=pltpu.PrefetchScalarGridSpec(
            num_scalar_prefetch=1,
            grid=(E,),
            in_specs=[
                pl.BlockSpec((PT, D), lambda e, sched: (0, 0)),
                pl.BlockSpec((1, D, H), lambda e, sched: (e, 0, 0)),
                pl.BlockSpec((1, 1, H), lambda e, sched: (e, 0, 0)),
                pl.BlockSpec((1, H, D), lambda e, sched: (e, 0, 0)),
                pl.BlockSpec((1, 1, D), lambda e, sched: (e, 0, 0)),
            ],
            out_specs=pl.BlockSpec((PT, D), lambda e, sched: (0, 0)),
        ),
        out_shape=jax.ShapeDtypeStruct((PT, D), jnp.bfloat16),
        compiler_params=pltpu.CompilerParams(
            dimension_semantics=("arbitrary",),
            vmem_limit_bytes=62 * 1024 * 1024),
    )(sched, xg.astype(jnp.bfloat16), W1,
      b1.reshape(E, 1, H), W2, b2.reshape(E, 1, D))

    out = pl.kernel(
        _combine_body,
        out_type=jax.ShapeDtypeStruct((S, D), jnp.float32),
        mesh=mesh,
        scratch_types=[
            pltpu.VMEM((TKW,), jnp.int32),
            pltpu.VMEM((TKW,), jnp.int32),
            pltpu.VMEM((TKW, 16), jnp.float32),
            pltpu.VMEM((TKW, 16), jnp.float32),
            pltpu.VMEM((TKW, D), jnp.float32),
            pltpu.VMEM((TKW, D), jnp.float32),
            pltpu.SemaphoreType.DMA,
        ],
    )(y.astype(jnp.float32), pos_flat, gateb)

    return out.reshape(x.shape)


# speedup vs baseline: 1.0147x; 1.0147x over previous
import jax
import jax.numpy as jnp
from jax import lax
from jax.experimental import pallas as pl
from jax.experimental.pallas import tpu as pltpu
from jax.experimental.pallas import tpu_sc as plsc

S = 2048
D = 768
E = 8
K = 2
H = 4 * D
A = S * K
BT = 128
NBP = A // BT + E - 1
PT = NBP * BT
NW = 32
CHW = A // NW
TKW = S // NW
CC = 256

_EPS = jax.random.normal(jax.random.key(42), (S, E), dtype=jnp.float32)
_EPS_T = _EPS.T


def _split64(v):
    hi = jnp.floor(v * (1.0 / 64.0))
    return hi, v - hi * 64.0


def _exact_dot_r(a, b):
    hi, lo = _split64(b)
    return (jnp.dot(a, hi, preferred_element_type=jnp.float32) * 64.0
            + jnp.dot(a, lo, preferred_element_type=jnp.float32))


def _router_kernel(zt_ref, epst_ref, z_ref, eps_ref,
                   pos_ref, gateb_ref, sched_ref):
    zt = zt_ref[...]
    noisyt = zt + epst_ref[...] * jax.nn.softplus(zt)
    idxe = lax.broadcasted_iota(jnp.int32, (E, S), 0)
    v0 = jnp.max(noisyt, axis=0, keepdims=True)
    i0 = jnp.min(jnp.where(noisyt == v0, idxe, E), axis=0, keepdims=True)
    m0 = idxe == i0
    masked = jnp.where(m0, -jnp.inf, noisyt)
    v1 = jnp.max(masked, axis=0, keepdims=True)
    i1 = jnp.min(jnp.where(masked == v1, idxe, E), axis=0, keepdims=True)
    m1 = idxe == i1

    oh0 = m0.astype(jnp.float32)
    oh1 = m1.astype(jnp.float32)
    counts_col = jnp.sum(oh0 + oh1, axis=1, keepdims=True)
    blocks_col = jnp.floor((counts_col + (BT - 1)) * (1.0 / BT))
    pad_col = blocks_col * BT
    tril = (lax.broadcasted_iota(jnp.int32, (E, E), 1)
            < lax.broadcasted_iota(jnp.int32, (E, E), 0)).astype(jnp.float32)
    offp_col = _exact_dot_r(tril, pad_col)

    up = (lax.broadcasted_iota(jnp.int32, (CC, CC), 0)
          < lax.broadcasted_iota(jnp.int32, (CC, CC), 1)).astype(jnp.float32)
    prefix = jnp.zeros((E, 1), jnp.float32)
    for slot, (oh, m) in enumerate(((oh0, m0), (oh1, m1))):
        for i in range(S // CC):
            blk = oh[:, i * CC:(i + 1) * CC]
            mblk = m[:, i * CC:(i + 1) * CC]
            rank = jnp.dot(blk, up, preferred_element_type=jnp.float32) + prefix
            dest = jnp.sum(jnp.where(mblk, rank + offp_col, 0.0),
                           axis=0, keepdims=True)
            pos_ref[slot:slot + 1, i * CC:(i + 1) * CC] = dest.astype(jnp.int32)
            prefix = prefix + jnp.sum(blk, axis=1, keepdims=True)

    z = z_ref[...]
    noisy = z + eps_ref[...] * jax.nn.softplus(z)
    idxe2 = lax.broadcasted_iota(jnp.int32, (S, E), 1)
    w0 = jnp.max(noisy, axis=1, keepdims=True)
    j0 = jnp.min(jnp.where(noisy == w0, idxe2, E), axis=1, keepdims=True)
    masked2 = jnp.where(idxe2 == j0, -jnp.inf, noisy)
    w1 = jnp.max(masked2, axis=1, keepdims=True)
    t = jnp.exp(w1 - w0)
    gateb_ref[0:S, :] = jnp.broadcast_to(1.0 / (1.0 + t), (S, 16))
    gateb_ref[S:2 * S, :] = jnp.broadcast_to(t / (1.0 + t), (S, 16))

    j1 = jnp.min(jnp.where(masked2 == w1, idxe2, E), axis=1, keepdims=True)
    counts_row = jnp.sum((idxe2 == j0).astype(jnp.float32)
                         + (idxe2 == j1).astype(jnp.float32),
                         axis=0, keepdims=True)
    blocks_row = jnp.floor((counts_row + (BT - 1)) * (1.0 / BT))
    triu_s = (lax.broadcasted_iota(jnp.int32, (E, E), 0)
              < lax.broadcasted_iota(jnp.int32, (E, E), 1)).astype(jnp.float32)
    bstart_row = jnp.dot(blocks_row, triu_s,
                         preferred_element_type=jnp.float32)
    total = jnp.sum(blocks_row, axis=1, keepdims=True)
    sched_ref[0:1, 0:E] = bstart_row.astype(jnp.int32)
    sched_ref[0:1, E:E + 1] = total.astype(jnp.int32)


def _gmm_kernel(sched_ref, xg_ref, w1_ref, b1_ref, w2_ref, b2_ref, out_ref):
    e = pl.program_id(0)
    lo = sched_ref[0, e]
    nb = sched_ref[0, e + 1] - lo

    def blk(j, carry):
        rb = (lo + j) * BT
        x = xg_ref[pl.ds(rb, BT), :]
        h = jnp.maximum(
            jnp.dot(x, w1_ref[0].astype(jnp.bfloat16),
                    preferred_element_type=jnp.float32)
            + b1_ref[0], 0.0).astype(jnp.bfloat16)
        o = jnp.dot(h, w2_ref[0].astype(jnp.bfloat16),
                    preferred_element_type=jnp.float32) + b2_ref[0]
        out_ref[pl.ds(rb, BT), :] = o.astype(jnp.bfloat16)
        return carry

    lax.fori_loop(0, nb, blk, 0)


def _dispatch_body(x_hbm, pos_hbm, xg_hbm, idx_v, xbuf, sem):
    c = lax.axis_index("c")
    sc = lax.axis_index("s")
    wid = sc * 2 + c
    tbase = (wid % 16) * CHW
    pltpu.sync_copy(x_hbm.at[pl.ds(tbase, CHW)], xbuf)
    pltpu.sync_copy(pos_hbm.at[pl.ds(wid * CHW, CHW)], idx_v)
    pltpu.async_copy(xbuf, xg_hbm.at[idx_v], sem).wait()


def _combine_body(y_hbm, pos_hbm, gateb_hbm, out_hbm,
                  i0_v, i1_v, g0_v, g1_v, buf0, buf1, sem):
    c = lax.axis_index("c")
    sc = lax.axis_index("s")
    wid = sc * 2 + c
    base = wid * TKW
    pltpu.sync_copy(pos_hbm.at[pl.ds(base, TKW)], i0_v)
    pltpu.sync_copy(pos_hbm.at[pl.ds(S + base, TKW)], i1_v)
    pltpu.sync_copy(gateb_hbm.at[pl.ds(base, TKW)], g0_v)
    pltpu.sync_copy(gateb_hbm.at[pl.ds(S + base, TKW)], g1_v)
    pltpu.async_copy(y_hbm.at[i0_v], buf0, sem).wait()
    pltpu.async_copy(y_hbm.at[i1_v], buf1, sem).wait()

    def row(r, carry):
        g0 = g0_v[r, pl.ds(0, 16)]
        g1 = g1_v[r, pl.ds(0, 16)]
        for j in range(D // 16):
            sl = pl.ds(j * 16, 16)
            buf0[r, sl] = buf0[r, sl] * g0 + buf1[r, sl] * g1
        return carry

    lax.fori_loop(0, TKW, row, 0)
    pltpu.sync_copy(buf0, out_hbm.at[pl.ds(base, TKW)])


def kernel(x, expert, W1, b1, W2, b2):
    flat_x = x.reshape(S, D)

    pos, gateb, sched = pl.pallas_call(
        _router_kernel,
        out_shape=[
            jax.ShapeDtypeStruct((K, S), jnp.int32),
            jax.ShapeDtypeStruct((K * S, 16), jnp.float32),
            jax.ShapeDtypeStruct((8, 128), jnp.int32),
        ],
    )(expert.T, _EPS_T, expert, _EPS)

    mesh = plsc.VectorSubcoreMesh(core_axis_name="c", subcore_axis_name="s")
    pos_flat = pos.reshape(A)

    xg = pl.kernel(
        _dispatch_body,
        out_type=jax.ShapeDtypeStruct((PT, D), jnp.float32),
        mesh=mesh,
        scratch_types=[
            pltpu.VMEM((CHW,), jnp.int32),
            pltpu.VMEM((CHW, D), jnp.float32),
            pltpu.SemaphoreType.DMA,
        ],
    )(flat_x, pos_flat)

    y = pl.pallas_call(
        _gmm_kernel,
        grid_spec=pltpu.PrefetchScalarGridSpec(
            num_scalar_prefetch=1,
            grid=(E,),
            in_specs=[
                pl.BlockSpec((PT, D), lambda e, sched: (0, 0)),
                pl.BlockSpec((1, D, H), lambda e, sched: (e, 0, 0)),
                pl.BlockSpec((1, 1, H), lambda e, sched: (e, 0, 0)),
                pl.BlockSpec((1, H, D), lambda e, sched: (e, 0, 0)),
                pl.BlockSpec((1, 1, D), lambda e, sched: (e, 0, 0)),
            ],
            out_specs=pl.BlockSpec((PT, D), lambda e, sched: (0, 0)),
        ),
        out_shape=jax.ShapeDtypeStruct((PT, D), jnp.bfloat16),
        compiler_params=pltpu.CompilerParams(
            dimension_semantics=("arbitrary",),
            vmem_limit_bytes=62 * 1024 * 1024),
    )(sched, xg.astype(jnp.bfloat16), W1,
      b1.reshape(E, 1, H), W2, b2.reshape(E, 1, D))

    out = pl.kernel(
        _combine_body,
        out_type=jax.ShapeDtypeStruct((S, D), jnp.float32),
        mesh=mesh,
        scratch_types=[
            pltpu.VMEM((TKW,), jnp.int32),
            pltpu.VMEM((TKW,), jnp.int32),
            pltpu.VMEM((TKW, 16), jnp.float32),
            pltpu.VMEM((TKW, 16), jnp.float32),
            pltpu.VMEM((TKW, D), jnp.float32),
            pltpu.VMEM((TKW, D), jnp.float32),
            pltpu.SemaphoreType.DMA,
        ],
    )(y.astype(jnp.float32), pos_flat, gateb)

    return out.reshape(x.shape)
